# baseline (speedup 1.0000x reference)
import functools
import math

import jax
import jax.numpy as jnp
from jax import lax
from jax.experimental import pallas as pl
from jax.experimental.pallas import tpu as pltpu
from jax.experimental.pallas import tpu_sc as plsc

N = 10000
E = 320000
H = 128
NH = 2
HD = H // NH
FF = 512

NP = 10240
TR = 512
TRA = 512
CB = 512
NPB = NP // TR

NSC = 2
NTI = 16
EPT = 10240
ECH = EPT // 128
EPAD = NSC * NTI * EPT

BN_SCALE = float(1.0 / math.sqrt(1.0 + 1e-5))
ATT_SCALE = float(1.0 / math.sqrt(HD))


R0 = 160
R1 = 0
PH = 8


@functools.cache
def _sc_scatter_kernel():
    return functools.partial(
        pl.kernel,
        out_type=jax.ShapeDtypeStruct((NSC * NP, H), jnp.float32),
        mesh=plsc.VectorSubcoreMesh(core_axis_name="c", subcore_axis_name="s"),
        scratch_types=[
            pltpu.VMEM_SHARED((NP, H), jnp.float32),
            pltpu.VMEM((PH, 128), jnp.int32),
            pltpu.VMEM((PH, 128), jnp.int32),
            pltpu.VMEM((128, H), jnp.float32),
            pltpu.VMEM((128, H), jnp.float32),
            *([pltpu.SemaphoreType.DMA] * 4),
        ],
    )(_sc_scatter_body)


def _sc_scatter_rows(x_hbm, src2, dst2):
    return _sc_scatter_kernel()(x_hbm, src2, dst2)


def _sc_scatter_body(x_hbm, src_hbm, dst_hbm, out_hbm, acc_sh, src_v, dst_v,
                     rows0, rows1, gsem0, gsem1, ssem0, ssem1):
    cid = lax.axis_index("c")
    if R1 == 0:
        @pl.when(cid == 0)
        def _():
            _sc_scatter_core(cid, x_hbm, src_hbm, dst_hbm, out_hbm, acc_sh,
                             src_v, dst_v, rows0, rows1, gsem0, gsem1,
                             ssem0, ssem1)
    else:
        _sc_scatter_core(cid, x_hbm, src_hbm, dst_hbm, out_hbm, acc_sh,
                         src_v, dst_v, rows0, rows1, gsem0, gsem1, ssem0,
                         ssem1)


def _sc_scatter_core(cid, x_hbm, src_hbm, dst_hbm, out_hbm, acc_sh, src_v,
                     dst_v, rows0, rows1, gsem0, gsem1, ssem0, ssem1):
    rows = (rows0, rows1)
    gsem = (gsem0, gsem1)
    ssem = (ssem0, ssem1)
    sid = lax.axis_index("s")
    rpt = NP // NTI
    row0 = sid * rpt

    zero16 = jnp.zeros((16,), jnp.float32)

    def _zero_row(i, carry):
        for j in range(H // 16):
            rows0[i, pl.ds(16 * j, 16)] = zero16
        return carry

    lax.fori_loop(0, 128, _zero_row, 0)

    for i in range(rpt // 128):
        pltpu.sync_copy(rows0, acc_sh.at[pl.ds(row0 + 128 * i, 128)])
    plsc.subcore_barrier()

    n_ph = jnp.where(cid == 0, R0 // PH, R1 // PH)
    t_base = jnp.where(cid == 0, sid * R0, NTI * R0 + sid * R1)

    def _issue_gather(c, b):
        pltpu.async_copy(x_hbm.at[src_v.at[c]], rows[b], gsem[b])

    def _wait_gather(b):
        pltpu.make_async_copy(x_hbm.at[src_v.at[0]], rows[b], gsem[b]).wait()

    def _issue_scatter(c, b):
        pltpu.async_copy(rows[b], acc_sh.at[dst_v.at[c]], ssem[b], add=True)

    def _wait_scatter(b):
        pltpu.make_async_copy(rows[b], acc_sh.at[dst_v.at[0]],
                              ssem[b]).wait()

    def _phase(p, carry):
        cb = t_base + p * PH
        pltpu.sync_copy(src_hbm.at[pl.ds(cb, PH)], src_v)
        pltpu.sync_copy(dst_hbm.at[pl.ds(cb, PH)], dst_v)
        _issue_gather(0, 0)
        _issue_gather(1, 1)
        _wait_gather(0)
        _issue_scatter(0, 0)

        def _step(g, inner):
            for b in range(2):
                c = 2 * g + b
                _wait_scatter(b)
                _issue_gather(c, b)
                b2 = 1 - b
                _wait_gather(b2)
                _issue_scatter(c - 1, b2)
            return inner

        lax.fori_loop(1, PH // 2, _step, 0)

        _wait_gather(1)
        _issue_scatter(PH - 1, 1)
        _wait_scatter(0)
        _wait_scatter(1)
        return carry

    lax.fori_loop(0, n_ph, _phase, 0)

    plsc.subcore_barrier()
    out0 = cid * NP + row0
    pltpu.sync_copy(acc_sh.at[pl.ds(row0, rpt)],
                    out_hbm.at[pl.ds(out0, rpt)])


def _dgT(a, w):
    return lax.dot_general(a, w, (((1,), (1,)), ((), ())),
                           preferred_element_type=jnp.float32)


def _layer_norm(a, g, b):
    m = jnp.mean(a, axis=-1, keepdims=True)
    d = a - m
    v = jnp.mean(d * d, axis=-1, keepdims=True)
    return d / jnp.sqrt(v + 1e-5) * g + b


def _mlp_body(x_ref, a0_ref, a1_ref, wa_ref, ba_ref, wb_ref, bb_ref, g_ref,
              be_ref, o_ref):
    t = x_ref[...] + a0_ref[...] + a1_ref[...]
    u = jnp.maximum(_dgT(t, wa_ref[...]) + ba_ref[...], 0.0)
    h = _dgT(u, wb_ref[...]) + bb_ref[...]
    o_ref[...] = jnp.maximum(h * BN_SCALE * g_ref[...] + be_ref[...], 0.0)


def _mlp(x, a0, a1, wa, ba, wb, bb, g, be):
    row = pl.BlockSpec((TR, H), lambda r: (r, 0))
    full = lambda shape: pl.BlockSpec(shape, lambda r: (0, 0))
    return pl.pallas_call(
        _mlp_body,
        grid=(NPB,),
        in_specs=[row, row, row, full((H, H)), full((1, H)), full((H, H)),
                  full((1, H)), full((1, H)), full((1, H))],
        out_specs=row,
        out_shape=jax.ShapeDtypeStruct((NP, H), jnp.float32),
        compiler_params=pltpu.CompilerParams(
            dimension_semantics=("parallel",)),
    )(x, a0, a1, wa, ba, wb, bb, g, be)


def _mlp_qkv_body(h_ref, a0_ref, a1_ref, wa_ref, ba_ref, wb_ref, bb_ref,
                  g_ref, be_ref, bpos_ref, wqkv_ref, bqkv_ref,
                  nodes_ref, q_ref, k_ref, v_ref):
    t = h_ref[...] + a0_ref[...] + a1_ref[...]
    u = jnp.maximum(_dgT(t, wa_ref[...]) + ba_ref[...], 0.0)
    h2 = _dgT(u, wb_ref[...]) + bb_ref[...]
    h2 = jnp.maximum(h2 * BN_SCALE * g_ref[...] + be_ref[...], 0.0)
    nodes = h2 + bpos_ref[...]
    qkv = _dgT(nodes, wqkv_ref[...]) + bqkv_ref[...]
    nodes_ref[...] = nodes
    q_ref[...] = qkv[:, 0:H]
    k_ref[...] = qkv[:, H:2 * H]
    v_ref[...] = qkv[:, 2 * H:3 * H]


def _mlp_qkv(h, a0, a1, wa, ba, wb, bb, g, be, bpos, wqkv, bqkv):
    row = pl.BlockSpec((TR, H), lambda r: (r, 0))
    full = lambda shape: pl.BlockSpec(shape, lambda r: (0, 0))
    outs = jax.ShapeDtypeStruct((NP, H), jnp.float32)
    return pl.pallas_call(
        _mlp_qkv_body,
        grid=(NPB,),
        in_specs=[row, row, row, full((H, H)), full((1, H)), full((H, H)),
                  full((1, H)), full((1, H)), full((1, H)), full((1, H)),
                  full((3 * H, H)), full((1, 3 * H))],
        out_specs=[row, row, row, row],
        out_shape=[outs, outs, outs, outs],
        compiler_params=pltpu.CompilerParams(
            dimension_semantics=("parallel",)),
    )(h, a0, a1, wa, ba, wb, bb, g, be, bpos, wqkv, bqkv)


def _attn_body(nodes_ref, q_ref, k_ref, v_ref, br_ref, bc_ref, wo_ref, bo_ref,
               lg1_ref, lb1_ref, wf1_ref, bf1_ref, wf2_ref, bf2_ref, lg2_ref,
               lb2_ref, o_ref):
    br = br_ref[...]
    bc = bc_ref[...]
    b_lo = jnp.min(br)
    b_hi = jnp.max(br)
    jstart = jnp.sum((bc < b_lo).astype(jnp.int32))
    jend = jnp.sum((bc <= b_hi).astype(jnp.int32))
    jb0 = jstart // CB
    jb1 = (jend + CB - 1) // CB

    q = q_ref[...] * ATT_SCALE
    outs = []
    for hh in range(NH):
        qh = q[:, hh * HD:(hh + 1) * HD]

        def body(j, carry, qh=qh, hh=hh):
            m, l, acc = carry
            off = pl.multiple_of(j * CB, CB)
            kh = k_ref[pl.ds(off, CB), :][:, hh * HD:(hh + 1) * HD]
            vh = v_ref[pl.ds(off, CB), :][:, hh * HD:(hh + 1) * HD]
            s = lax.dot_general(qh, kh, (((1,), (1,)), ((), ())),
                                preferred_element_type=jnp.float32)
            bcj = bc_ref[:, pl.ds(off, CB)]
            msk = br == bcj
            s = jnp.where(msk, s, -1e30)
            mnew = jnp.maximum(m, jnp.max(s, axis=1, keepdims=True))
            p = jnp.where(msk, jnp.exp(s - mnew), 0.0)
            corr = jnp.exp(m - mnew)
            l2 = l * corr + jnp.sum(p, axis=1, keepdims=True)
            acc2 = acc * corr + lax.dot_general(
                p, vh, (((1,), (0,)), ((), ())),
                preferred_element_type=jnp.float32)
            return mnew, l2, acc2

        init = (jnp.full((TRA, 1), -1e30, jnp.float32),
                jnp.zeros((TRA, 1), jnp.float32),
                jnp.zeros((TRA, HD), jnp.float32))
        m, l, acc = lax.fori_loop(jb0, jb1, body, init)
        outs.append(acc / l)

    o = jnp.concatenate(outs, axis=1)
    attn = _dgT(o, wo_ref[...]) + bo_ref[...]
    r1 = _layer_norm(nodes_ref[...] + attn, lg1_ref[...], lb1_ref[...])
    f = jnp.maximum(_dgT(r1, wf1_ref[...]) + bf1_ref[...], 0.0)
    f = _dgT(f, wf2_ref[...]) + bf2_ref[...]
    o_ref[...] = _layer_norm(r1 + f, lg2_ref[...], lb2_ref[...])


def _attn(nodes, q, k, v, br, bc, wo, bo, lg1, lb1, wf1, bf1, wf2, bf2, lg2,
          lb2):
    row = pl.BlockSpec((TRA, H), lambda r: (r, 0))
    full = lambda shape: pl.BlockSpec(shape, lambda r: (0, 0))
    return pl.pallas_call(
        _attn_body,
        grid=(NP // TRA,),
        in_specs=[row, row, full((NP, H)), full((NP, H)),
                  pl.BlockSpec((TRA, 1), lambda r: (r, 0)), full((1, NP)),
                  full((H, H)), full((1, H)), full((1, H)), full((1, H)),
                  full((FF, H)), full((1, FF)), full((H, FF)), full((1, H)),
                  full((1, H)), full((1, H))],
        out_specs=row,
        out_shape=jax.ShapeDtypeStruct((NP, H), jnp.float32),
        compiler_params=pltpu.CompilerParams(
            dimension_semantics=("arbitrary",)),
    )(nodes, q, k, v, br, bc, wo, bo, lg1, lb1, wf1, bf1, wf2, bf2, lg2, lb2)


def kernel(x, edge_index, batch, W1a, b1a, W1b, b1b, g1, be1, W2a, b2a, W2b,
           b2b, g2, be2, Wpos, bpos, Wqkv, bqkv, Wo, bo, lng1, lnb1, Wf1,
           bf1, Wf2, bf2, lng2, lnb2):
    x_p = jnp.pad(x, ((0, NP - N), (0, 0)))
    batch_i = batch.astype(jnp.int32)
    batch_p = jnp.pad(batch_i, (0, NP - N),
                      constant_values=jnp.iinfo(jnp.int32).max)
    br = batch_p.reshape(NP, 1)
    bc = batch_p.reshape(1, NP)
    src = edge_index[0].astype(jnp.int32)
    dst = edge_index[1].astype(jnp.int32)
    src2 = jnp.pad(src, (0, EPAD - E)).reshape(EPAD // 128, 128)
    dst2 = jnp.pad(dst, (0, EPAD - E), constant_values=N).reshape(
        EPAD // 128, 128)

    r1 = lambda a: a.reshape(1, -1)

    zagg = jnp.zeros((NP, H), jnp.float32)
    aggp = _sc_scatter_rows(x_p, src2, dst2)
    a1 = aggp[NP:] if R1 > 0 else zagg
    h = _mlp(x_p, aggp[:NP], a1, W1a, r1(b1a), W1b, r1(b1b), r1(g1),
             r1(be1))
    agg2p = _sc_scatter_rows(h, src2, dst2)
    a2 = agg2p[NP:] if R1 > 0 else zagg
    nodes, q, k, v = _mlp_qkv(h, agg2p[:NP], a2, W2a, r1(b2a), W2b,
                              r1(b2b), r1(g2), r1(be2), r1(bpos), Wqkv,
                              r1(bqkv))
    out = _attn(nodes, q, k, v, br, bc, Wo, r1(bo), r1(lng1), r1(lnb1), Wf1,
                r1(bf1), Wf2, r1(bf2), r1(lng2), r1(lnb2))
    return out[:N]


# speedup vs baseline: 1.5087x; 1.5087x over previous
import functools
import math

import jax
import jax.numpy as jnp
from jax import lax
from jax.experimental import pallas as pl
from jax.experimental.pallas import tpu as pltpu
from jax.experimental.pallas import tpu_sc as plsc

N = 10000
E = 320000
H = 128
NH = 2
HD = H // NH
FF = 512

NP = 10240
TR = 512
TRA = 512
CB = 512
NPB = NP // TR

NSC = 2
NTI = 16
EPT = 10240
ECH = EPT // 128
EPAD = NSC * NTI * EPT

BN_SCALE = float(1.0 / math.sqrt(1.0 + 1e-5))
ATT_SCALE = float(1.0 / math.sqrt(HD))


R0 = 152
R1 = 8
PH = 8


@functools.cache
def _sc_scatter_kernel():
    return functools.partial(
        pl.kernel,
        out_type=jax.ShapeDtypeStruct((NSC * NP, H), jnp.float32),
        mesh=plsc.VectorSubcoreMesh(core_axis_name="c", subcore_axis_name="s"),
        scratch_types=[
            pltpu.VMEM_SHARED((NP, H), jnp.float32),
            pltpu.VMEM((PH, 128), jnp.int32),
            pltpu.VMEM((PH, 128), jnp.int32),
            pltpu.VMEM((128, H), jnp.float32),
            pltpu.VMEM((128, H), jnp.float32),
            *([pltpu.SemaphoreType.DMA] * 4),
        ],
    )(_sc_scatter_body)


def _sc_scatter_rows(x_hbm, src2, dst2):
    return _sc_scatter_kernel()(x_hbm, src2, dst2)


def _sc_scatter_body(x_hbm, src_hbm, dst_hbm, out_hbm, acc_sh, src_v, dst_v,
                     rows0, rows1, gsem0, gsem1, ssem0, ssem1):
    cid = lax.axis_index("c")
    if R1 == 0:
        @pl.when(cid == 0)
        def _():
            _sc_scatter_core(cid, x_hbm, src_hbm, dst_hbm, out_hbm, acc_sh,
                             src_v, dst_v, rows0, rows1, gsem0, gsem1,
                             ssem0, ssem1)
    else:
        _sc_scatter_core(cid, x_hbm, src_hbm, dst_hbm, out_hbm, acc_sh,
                         src_v, dst_v, rows0, rows1, gsem0, gsem1, ssem0,
                         ssem1)


def _sc_scatter_core(cid, x_hbm, src_hbm, dst_hbm, out_hbm, acc_sh, src_v,
                     dst_v, rows0, rows1, gsem0, gsem1, ssem0, ssem1):
    rows = (rows0, rows1)
    gsem = (gsem0, gsem1)
    ssem = (ssem0, ssem1)
    sid = lax.axis_index("s")
    rpt = NP // NTI
    row0 = sid * rpt

    zero16 = jnp.zeros((16,), jnp.float32)

    def _zero_row(i, carry):
        for j in range(H // 16):
            rows0[i, pl.ds(16 * j, 16)] = zero16
        return carry

    lax.fori_loop(0, 128, _zero_row, 0)

    for i in range(rpt // 128):
        pltpu.sync_copy(rows0, acc_sh.at[pl.ds(row0 + 128 * i, 128)])
    plsc.subcore_barrier()

    n_ph = jnp.where(cid == 0, R0 // PH, R1 // PH)
    t_base = jnp.where(cid == 0, sid * R0, NTI * R0 + sid * R1)

    def _issue_gather(c, b):
        pltpu.async_copy(x_hbm.at[src_v.at[c]], rows[b], gsem[b])

    def _wait_gather(b):
        pltpu.make_async_copy(x_hbm.at[src_v.at[0]], rows[b], gsem[b]).wait()

    def _issue_scatter(c, b):
        pltpu.async_copy(rows[b], acc_sh.at[dst_v.at[c]], ssem[b], add=True)

    def _wait_scatter(b):
        pltpu.make_async_copy(rows[b], acc_sh.at[dst_v.at[0]],
                              ssem[b]).wait()

    def _phase(p, carry):
        cb = t_base + p * PH
        pltpu.sync_copy(src_hbm.at[pl.ds(cb, PH)], src_v)
        pltpu.sync_copy(dst_hbm.at[pl.ds(cb, PH)], dst_v)
        _issue_gather(0, 0)
        _issue_gather(1, 1)
        _wait_gather(0)
        _issue_scatter(0, 0)

        def _step(g, inner):
            for b in range(2):
                c = 2 * g + b
                _wait_scatter(b)
                _issue_gather(c, b)
                b2 = 1 - b
                _wait_gather(b2)
                _issue_scatter(c - 1, b2)
            return inner

        lax.fori_loop(1, PH // 2, _step, 0)

        _wait_gather(1)
        _issue_scatter(PH - 1, 1)
        _wait_scatter(0)
        _wait_scatter(1)
        return carry

    lax.fori_loop(0, n_ph, _phase, 0)

    plsc.subcore_barrier()
    out0 = cid * NP + row0
    pltpu.sync_copy(acc_sh.at[pl.ds(row0, rpt)],
                    out_hbm.at[pl.ds(out0, rpt)])


def _dgT(a, w):
    return lax.dot_general(a, w, (((1,), (1,)), ((), ())),
                           preferred_element_type=jnp.float32)


def _layer_norm(a, g, b):
    m = jnp.mean(a, axis=-1, keepdims=True)
    d = a - m
    v = jnp.mean(d * d, axis=-1, keepdims=True)
    return d / jnp.sqrt(v + 1e-5) * g + b


def _mlp_body(x_ref, a0_ref, a1_ref, wa_ref, ba_ref, wb_ref, bb_ref, g_ref,
              be_ref, o_ref):
    t = x_ref[...] + a0_ref[...] + a1_ref[...]
    u = jnp.maximum(_dgT(t, wa_ref[...]) + ba_ref[...], 0.0)
    h = _dgT(u, wb_ref[...]) + bb_ref[...]
    o_ref[...] = jnp.maximum(h * BN_SCALE * g_ref[...] + be_ref[...], 0.0)


def _mlp(x, a0, a1, wa, ba, wb, bb, g, be):
    row = pl.BlockSpec((TR, H), lambda r: (r, 0))
    full = lambda shape: pl.BlockSpec(shape, lambda r: (0, 0))
    return pl.pallas_call(
        _mlp_body,
        grid=(NPB,),
        in_specs=[row, row, row, full((H, H)), full((1, H)), full((H, H)),
                  full((1, H)), full((1, H)), full((1, H))],
        out_specs=row,
        out_shape=jax.ShapeDtypeStruct((NP, H), jnp.float32),
        compiler_params=pltpu.CompilerParams(
            dimension_semantics=("parallel",)),
    )(x, a0, a1, wa, ba, wb, bb, g, be)


def _mlp_qkv_body(h_ref, a0_ref, a1_ref, wa_ref, ba_ref, wb_ref, bb_ref,
                  g_ref, be_ref, bpos_ref, wqkv_ref, bqkv_ref,
                  nodes_ref, q_ref, k_ref, v_ref):
    t = h_ref[...] + a0_ref[...] + a1_ref[...]
    u = jnp.maximum(_dgT(t, wa_ref[...]) + ba_ref[...], 0.0)
    h2 = _dgT(u, wb_ref[...]) + bb_ref[...]
    h2 = jnp.maximum(h2 * BN_SCALE * g_ref[...] + be_ref[...], 0.0)
    nodes = h2 + bpos_ref[...]
    qkv = _dgT(nodes, wqkv_ref[...]) + bqkv_ref[...]
    nodes_ref[...] = nodes
    q_ref[...] = qkv[:, 0:H]
    k_ref[...] = qkv[:, H:2 * H]
    v_ref[...] = qkv[:, 2 * H:3 * H]


def _mlp_qkv(h, a0, a1, wa, ba, wb, bb, g, be, bpos, wqkv, bqkv):
    row = pl.BlockSpec((TR, H), lambda r: (r, 0))
    full = lambda shape: pl.BlockSpec(shape, lambda r: (0, 0))
    outs = jax.ShapeDtypeStruct((NP, H), jnp.float32)
    return pl.pallas_call(
        _mlp_qkv_body,
        grid=(NPB,),
        in_specs=[row, row, row, full((H, H)), full((1, H)), full((H, H)),
                  full((1, H)), full((1, H)), full((1, H)), full((1, H)),
                  full((3 * H, H)), full((1, 3 * H))],
        out_specs=[row, row, row, row],
        out_shape=[outs, outs, outs, outs],
        compiler_params=pltpu.CompilerParams(
            dimension_semantics=("parallel",)),
    )(h, a0, a1, wa, ba, wb, bb, g, be, bpos, wqkv, bqkv)


def _attn_body(nodes_ref, q_ref, k_ref, v_ref, br_ref, bc_ref, wo_ref, bo_ref,
               lg1_ref, lb1_ref, wf1_ref, bf1_ref, wf2_ref, bf2_ref, lg2_ref,
               lb2_ref, o_ref):
    br = br_ref[...]
    bc = bc_ref[...]
    b_lo = jnp.min(br)
    b_hi = jnp.max(br)
    jstart = jnp.sum((bc < b_lo).astype(jnp.int32))
    jend = jnp.sum((bc <= b_hi).astype(jnp.int32))
    jb0 = jstart // CB
    jb1 = (jend + CB - 1) // CB

    q = q_ref[...] * ATT_SCALE
    outs = []
    for hh in range(NH):
        qh = q[:, hh * HD:(hh + 1) * HD]

        def body(j, carry, qh=qh, hh=hh):
            m, l, acc = carry
            off = pl.multiple_of(j * CB, CB)
            kh = k_ref[pl.ds(off, CB), :][:, hh * HD:(hh + 1) * HD]
            vh = v_ref[pl.ds(off, CB), :][:, hh * HD:(hh + 1) * HD]
            s = lax.dot_general(qh, kh, (((1,), (1,)), ((), ())),
                                preferred_element_type=jnp.float32)
            bcj = bc_ref[:, pl.ds(off, CB)]
            msk = br == bcj
            s = jnp.where(msk, s, -1e30)
            mnew = jnp.maximum(m, jnp.max(s, axis=1, keepdims=True))
            p = jnp.where(msk, jnp.exp(s - mnew), 0.0)
            corr = jnp.exp(m - mnew)
            l2 = l * corr + jnp.sum(p, axis=1, keepdims=True)
            acc2 = acc * corr + lax.dot_general(
                p, vh, (((1,), (0,)), ((), ())),
                preferred_element_type=jnp.float32)
            return mnew, l2, acc2

        init = (jnp.full((TRA, 1), -1e30, jnp.float32),
                jnp.zeros((TRA, 1), jnp.float32),
                jnp.zeros((TRA, HD), jnp.float32))
        m, l, acc = lax.fori_loop(jb0, jb1, body, init)
        outs.append(acc / l)

    o = jnp.concatenate(outs, axis=1)
    attn = _dgT(o, wo_ref[...]) + bo_ref[...]
    r1 = _layer_norm(nodes_ref[...] + attn, lg1_ref[...], lb1_ref[...])
    f = jnp.maximum(_dgT(r1, wf1_ref[...]) + bf1_ref[...], 0.0)
    f = _dgT(f, wf2_ref[...]) + bf2_ref[...]
    o_ref[...] = _layer_norm(r1 + f, lg2_ref[...], lb2_ref[...])


def _attn(nodes, q, k, v, br, bc, wo, bo, lg1, lb1, wf1, bf1, wf2, bf2, lg2,
          lb2):
    row = pl.BlockSpec((TRA, H), lambda r: (r, 0))
    full = lambda shape: pl.BlockSpec(shape, lambda r: (0, 0))
    return pl.pallas_call(
        _attn_body,
        grid=(NP // TRA,),
        in_specs=[row, row, full((NP, H)), full((NP, H)),
                  pl.BlockSpec((TRA, 1), lambda r: (r, 0)), full((1, NP)),
                  full((H, H)), full((1, H)), full((1, H)), full((1, H)),
                  full((FF, H)), full((1, FF)), full((H, FF)), full((1, H)),
                  full((1, H)), full((1, H))],
        out_specs=row,
        out_shape=jax.ShapeDtypeStruct((NP, H), jnp.float32),
        compiler_params=pltpu.CompilerParams(
            dimension_semantics=("arbitrary",)),
    )(nodes, q, k, v, br, bc, wo, bo, lg1, lb1, wf1, bf1, wf2, bf2, lg2, lb2)


def kernel(x, edge_index, batch, W1a, b1a, W1b, b1b, g1, be1, W2a, b2a, W2b,
           b2b, g2, be2, Wpos, bpos, Wqkv, bqkv, Wo, bo, lng1, lnb1, Wf1,
           bf1, Wf2, bf2, lng2, lnb2):
    x_p = jnp.pad(x, ((0, NP - N), (0, 0)))
    batch_i = batch.astype(jnp.int32)
    batch_p = jnp.pad(batch_i, (0, NP - N),
                      constant_values=jnp.iinfo(jnp.int32).max)
    br = batch_p.reshape(NP, 1)
    bc = batch_p.reshape(1, NP)
    src = edge_index[0].astype(jnp.int32)
    dst = edge_index[1].astype(jnp.int32)
    src2 = jnp.pad(src, (0, EPAD - E)).reshape(EPAD // 128, 128)
    dst2 = jnp.pad(dst, (0, EPAD - E), constant_values=N).reshape(
        EPAD // 128, 128)

    r1 = lambda a: a.reshape(1, -1)

    zagg = jnp.zeros((NP, H), jnp.float32)
    aggp = _sc_scatter_rows(x_p, src2, dst2)
    a1 = aggp[NP:] if R1 > 0 else zagg
    h = _mlp(x_p, aggp[:NP], a1, W1a, r1(b1a), W1b, r1(b1b), r1(g1),
             r1(be1))
    agg2p = _sc_scatter_rows(h, src2, dst2)
    a2 = agg2p[NP:] if R1 > 0 else zagg
    nodes, q, k, v = _mlp_qkv(h, agg2p[:NP], a2, W2a, r1(b2a), W2b,
                              r1(b2b), r1(g2), r1(be2), r1(bpos), Wqkv,
                              r1(bqkv))
    out = _attn(nodes, q, k, v, br, bc, Wo, r1(bo), r1(lng1), r1(lnb1), Wf1,
                r1(bf1), Wf2, r1(bf2), r1(lng2), r1(lnb2))
    return out[:N]
